# baseline (speedup 1.0000x reference)
import functools

import jax
import jax.numpy as jnp
from jax import lax
from jax.experimental import pallas as pl
from jax.experimental.pallas import tpu as pltpu
from jax.experimental.pallas import tpu_sc as plsc

DIM = 16
CTX = 50
LANES = 16
NUM_CORES = 2
NUM_SUBCORES = 16
NUM_WORKERS = NUM_CORES * NUM_SUBCORES


def _make_kernel(B, V):
    b_per_w = B // NUM_WORKERS
    n_groups = b_per_w // LANES
    rows_per_group = LANES * CTX
    chunks = [(o, min(128, rows_per_group - o))
              for o in range(0, rows_per_group, 128)]

    mesh = plsc.VectorSubcoreMesh(core_axis_name="c", subcore_axis_name="s")

    @functools.partial(
        pl.kernel,
        out_type=jax.ShapeDtypeStruct((B * CTX,), jnp.float32),
        mesh=mesh,
        compiler_params=pltpu.CompilerParams(
            needs_layout_passes=False, use_tc_tiling_on_sc=False),
        scratch_types=[
            pltpu.VMEM((b_per_w,), jnp.int32),
            pltpu.VMEM((b_per_w * CTX,), jnp.int32),
            pltpu.VMEM((b_per_w, DIM), jnp.float32),
            pltpu.VMEM((rows_per_group, DIM), jnp.float32),
            pltpu.VMEM((rows_per_group, DIM), jnp.float32),
            pltpu.VMEM((rows_per_group,), jnp.float32),
            pltpu.VMEM((rows_per_group,), jnp.float32),
            pltpu.SemaphoreType.DMA,
            pltpu.SemaphoreType.DMA,
            pltpu.SemaphoreType.DMA,
            pltpu.SemaphoreType.DMA,
        ],
    )
    def word_embed(center_hbm, context_hbm, wc_hbm, wx_hbm, out_hbm,
                   cidx_v, xidx_v, crows_v, xrows0, xrows1, outv0, outv1,
                   gs0, gs1, os0, os1):
        wid = lax.axis_index("s") * NUM_CORES + lax.axis_index("c")
        base = wid * b_per_w
        xrows = (xrows0, xrows1)
        outvs = (outv0, outv1)
        gsems = (gs0, gs1)
        osems = (os0, os1)

        pltpu.sync_copy(center_hbm.at[pl.ds(base, b_per_w)], cidx_v)
        pltpu.sync_copy(context_hbm.at[pl.ds(base * CTX, b_per_w * CTX)],
                        xidx_v)
        pltpu.sync_copy(wc_hbm.at[cidx_v], crows_v)

        def fire_group(g, buf):
            descs = []
            for off, n in chunks:
                idx = xidx_v.at[pl.ds(g * rows_per_group + off, n)]
                descs.append(pltpu.async_copy(
                    wx_hbm.at[idx], xrows[buf].at[pl.ds(off, n)],
                    gsems[buf]))
            return descs

        lane = lax.iota(jnp.int32, LANES)

        def compute_group(g, buf):
            rows = xrows[buf]
            ov = outvs[buf]
            ccols = [plsc.load_gather(
                crows_v, [g * LANES + lane, jnp.full((LANES,), d, jnp.int32)])
                for d in range(DIM)]

            def body(m, _):
                row_idx = lane * CTX + m
                acc = jnp.zeros((LANES,), jnp.float32)
                for d in range(DIM):
                    xcol = plsc.load_gather(
                        rows, [row_idx, jnp.full((LANES,), d, jnp.int32)])
                    acc = acc + ccols[d] * xcol
                plsc.store_scatter(ov, [row_idx], acc)
                return _

            lax.fori_loop(0, CTX, body, 0, unroll=2)

        inflight = {}
        out_descs = {}
        inflight[0] = fire_group(0, 0)
        for g in range(n_groups):
            buf = g % 2
            if g + 1 < n_groups:
                inflight[g + 1] = fire_group(g + 1, (g + 1) % 2)
            for dsc in inflight.pop(g):
                dsc.wait()
            if g - 2 in out_descs:
                out_descs.pop(g - 2).wait()
            compute_group(g, buf)
            out_descs[g] = pltpu.async_copy(
                outvs[buf],
                out_hbm.at[pl.ds((base + g * LANES) * CTX, rows_per_group)],
                osems[buf])
        for dsc in out_descs.values():
            dsc.wait()

    return word_embed


def kernel(center, context, W_center, W_context):
    B = center.shape[0]
    V = W_center.shape[0]
    k = _make_kernel(B, V)
    out_flat = k(center.reshape(B).astype(jnp.int32),
                 context.reshape(B * CTX).astype(jnp.int32),
                 W_center, W_context)
    return out_flat.reshape(B, 1, CTX)
